# baseline (speedup 1.0000x reference)
import functools

import jax
import jax.numpy as jnp
from jax import lax
from jax.experimental import pallas as pl
from jax.experimental.pallas import tpu as pltpu
from jax.experimental.pallas import tpu_sc as plsc

_B, _S, _H = 4, 8192, 2048
_L = 16


def _entity_gather(pos_hbm, table_hbm, out1_hbm, out2_hbm, idx_v, rows_v, sem):
    cid = lax.axis_index("c")
    sid = lax.axis_index("s")

    @pl.when(jnp.logical_and(cid == 0, sid == 0))
    def _():
        pltpu.sync_copy(pos_hbm, idx_v)
        pos = idx_v[...]
        b = lax.rem(lax.iota(jnp.int32, _L), _B)
        idx_v[...] = pos + b * _S
        pltpu.async_copy(
            table_hbm.at[idx_v.at[pl.ds(0, 2 * _B)]], rows_v, sem
        ).wait()
        pltpu.sync_copy(rows_v.at[pl.ds(0, _B)], out1_hbm)
        pltpu.sync_copy(rows_v.at[pl.ds(_B, _B)], out2_hbm)


@jax.jit
def _run(table, posflat):
    mesh = plsc.VectorSubcoreMesh(core_axis_name="c", subcore_axis_name="s")
    f = functools.partial(
        pl.kernel,
        mesh=mesh,
        out_type=(
            jax.ShapeDtypeStruct((_B, _H), jnp.float32),
            jax.ShapeDtypeStruct((_B, _H), jnp.float32),
        ),
        scratch_types=[
            pltpu.VMEM((_L,), jnp.int32),
            pltpu.VMEM((2 * _B, _H), jnp.float32),
            pltpu.SemaphoreType.DMA,
        ],
    )(_entity_gather)
    return f(posflat, table)


def kernel(token_embs, pos1, pos2, mask):
    B, S, H = token_embs.shape
    table = token_embs.reshape(B * S, H)
    posflat = jnp.concatenate(
        [pos1[:, 0], pos2[:, 0], jnp.zeros((2 * _L - 2 * B,), pos1.dtype)]
    ).astype(jnp.int32)[: _L]
    return _run(table, posflat)


# speedup vs baseline: 1.0805x; 1.0805x over previous
import functools

import jax
import jax.numpy as jnp
from jax import lax
from jax.experimental import pallas as pl
from jax.experimental.pallas import tpu as pltpu
from jax.experimental.pallas import tpu_sc as plsc

_B, _S, _H = 4, 8192, 2048


def _entity_gather(pos_hbm, table_hbm, out1_hbm, out2_hbm, pos_smem, sem):
    cid = lax.axis_index("c")

    @pl.when(cid == 0)
    def _():
        pltpu.sync_copy(pos_hbm, pos_smem)
        for b in range(_B):
            r1 = pos_smem[b] + b * _S
            pltpu.async_copy(
                table_hbm.at[pl.ds(r1, 1)], out1_hbm.at[pl.ds(b, 1)], sem
            )
            r2 = pos_smem[_B + b] + b * _S
            pltpu.async_copy(
                table_hbm.at[pl.ds(r2, 1)], out2_hbm.at[pl.ds(b, 1)], sem
            )
        for b in range(_B):
            pltpu.make_async_copy(
                table_hbm.at[pl.ds(0, 1)], out1_hbm.at[pl.ds(b, 1)], sem
            ).wait()
            pltpu.make_async_copy(
                table_hbm.at[pl.ds(0, 1)], out2_hbm.at[pl.ds(b, 1)], sem
            ).wait()


@jax.jit
def _run(table, posflat):
    mesh = plsc.ScalarSubcoreMesh(axis_name="c")
    f = functools.partial(
        pl.kernel,
        mesh=mesh,
        out_type=(
            jax.ShapeDtypeStruct((_B, _H), jnp.float32),
            jax.ShapeDtypeStruct((_B, _H), jnp.float32),
        ),
        scratch_types=[
            pltpu.SMEM((2 * _B,), jnp.int32),
            pltpu.SemaphoreType.DMA,
        ],
    )(_entity_gather)
    return f(posflat, table)


def kernel(token_embs, pos1, pos2, mask):
    B, S, H = token_embs.shape
    table = token_embs.reshape(B * S, H)
    posflat = jnp.concatenate([pos1[:, 0], pos2[:, 0]]).astype(jnp.int32)
    return _run(table, posflat)


# speedup vs baseline: 1.1551x; 1.0690x over previous
import functools

import jax
import jax.numpy as jnp
from jax import lax
from jax.experimental import pallas as pl
from jax.experimental.pallas import tpu as pltpu
from jax.experimental.pallas import tpu_sc as plsc

_B, _S, _H = 4, 8192, 2048


def _entity_gather(pos_hbm, table_hbm, out1_hbm, out2_hbm, pos_smem, sem):
    cid = lax.axis_index("c")

    @pl.when(cid == 0)
    def _():
        pltpu.sync_copy(pos_hbm, pos_smem)
        for b in range(_B):
            r1 = pos_smem[b] + b * _S
            pltpu.async_copy(
                table_hbm.at[pl.ds(r1, 1)], out1_hbm.at[pl.ds(b, 1)], sem
            )
            r2 = pos_smem[_B + b] + b * _S
            pltpu.async_copy(
                table_hbm.at[pl.ds(r2, 1)], out2_hbm.at[pl.ds(b, 1)], sem
            )
        for b in range(_B):
            pltpu.make_async_copy(
                table_hbm.at[pl.ds(0, 1)], out1_hbm.at[pl.ds(b, 1)], sem
            ).wait()
            pltpu.make_async_copy(
                table_hbm.at[pl.ds(0, 1)], out2_hbm.at[pl.ds(b, 1)], sem
            ).wait()


@jax.jit
def _run(table, posflat):
    mesh = plsc.ScalarSubcoreMesh(axis_name="c", num_cores=1)
    f = functools.partial(
        pl.kernel,
        mesh=mesh,
        out_type=(
            jax.ShapeDtypeStruct((_B, _H), jnp.float32),
            jax.ShapeDtypeStruct((_B, _H), jnp.float32),
        ),
        scratch_types=[
            pltpu.SMEM((2 * _B,), jnp.int32),
            pltpu.SemaphoreType.DMA,
        ],
    )(_entity_gather)
    return f(posflat, table)


def kernel(token_embs, pos1, pos2, mask):
    B, S, H = token_embs.shape
    table = token_embs.reshape(B * S, H)
    posflat = jnp.concatenate([pos1[:, 0], pos2[:, 0]]).astype(jnp.int32)
    return _run(table, posflat)
